# baseline (speedup 1.0000x reference)
import functools

import jax
import jax.numpy as jnp
from jax import lax
from jax.experimental import pallas as pl
from jax.experimental.pallas import tpu as pltpu
from jax.experimental.pallas import tpu_sc as plsc

_B, _C, _K, _T = 64, 32, 1024, 1024
_NC, _NS, _L = 2, 16, 16
_NW = _NC * _NS
_BPW = _B // _NW


def _vq_idx_body(x_ref, w_ref, idx_ref, wn_ref):
    @pl.when(pl.program_id(0) == 0)
    def _():
        w = w_ref[...]
        wn_ref[...] = 0.5 * jnp.sum(w * w, axis=1, keepdims=True)

    xb = x_ref[0]
    prod = jnp.dot(w_ref[...], xb,
                   preferred_element_type=jnp.float32)
    scores = wn_ref[...] - prod
    idx_ref[0, 0] = jnp.argmin(scores, axis=0).astype(jnp.int32)


def _sc_gather(wflat_hbm, idx_hbm, codes_hbm, w_v, idx_v, codes_v):
    wid = lax.axis_index("s") * _NC + lax.axis_index("c")
    b0 = wid * _BPW
    pltpu.sync_copy(wflat_hbm, w_v)
    pltpu.sync_copy(idx_hbm.at[pl.ds(b0, _BPW)], idx_v)

    @plsc.parallel_loop(0, _T // _L, unroll=2)
    def _body(g):
        sl = pl.ds(g * _L, _L)
        for bb in range(_BPW):
            base = idx_v[bb, sl] * _C
            for c in range(_C):
                codes_v[bb, c, sl] = plsc.load_gather(w_v, [base + c])

    pltpu.sync_copy(codes_v, codes_hbm.at[pl.ds(b0, _BPW)])


def kernel(x, W):
    xr = x.reshape(_B, _C, _T)
    idx3 = pl.pallas_call(
        _vq_idx_body,
        grid=(_B,),
        in_specs=[
            pl.BlockSpec((1, _C, _T), lambda b: (b, 0, 0)),
            pl.BlockSpec((_K, _C), lambda b: (0, 0)),
        ],
        out_specs=pl.BlockSpec((1, 1, _T), lambda b: (b, 0, 0)),
        out_shape=jax.ShapeDtypeStruct((_B, 1, _T), jnp.int32),
        scratch_shapes=[pltpu.VMEM((_K, 1), jnp.float32)],
    )(xr, W)
    idx2 = idx3.reshape(_B, _T)

    gather = functools.partial(
        pl.kernel,
        out_type=jax.ShapeDtypeStruct((_B, _C, _T), jnp.float32),
        mesh=plsc.VectorSubcoreMesh(core_axis_name="c", subcore_axis_name="s"),
        scratch_types=[
            pltpu.VMEM((_C * _K,), jnp.float32),
            pltpu.VMEM((_BPW, _T), jnp.int32),
            pltpu.VMEM((_BPW, _C, _T), jnp.float32),
        ],
        compiler_params=pltpu.CompilerParams(
            needs_layout_passes=False, use_tc_tiling_on_sc=False),
    )(_sc_gather)
    codes3 = gather(W.reshape(-1), idx2)

    codes = codes3.reshape(x.shape)
    indices = idx2.reshape(_B, 32, 32)
    return codes, indices


# speedup vs baseline: 1.0776x; 1.0776x over previous
import functools

import jax
import jax.numpy as jnp
from jax import lax
from jax.experimental import pallas as pl
from jax.experimental.pallas import tpu as pltpu
from jax.experimental.pallas import tpu_sc as plsc

_B, _C, _K, _T = 64, 32, 1024, 1024
_NC, _NS, _L = 2, 16, 16
_NW = _NC * _NS
_BPW = _B // _NW


def _vq_idx_body(x_ref, w_ref, idx_ref, wn_ref):
    @pl.when(pl.program_id(0) == 0)
    def _():
        w = w_ref[...]
        wn_ref[...] = 0.5 * jnp.sum(w * w, axis=1, keepdims=True)

    for i in range(2):
        xb = x_ref[i]
        prod = jnp.dot(w_ref[...], xb,
                       preferred_element_type=jnp.float32)
        scores = wn_ref[...] - prod
        idx_ref[i, 0] = jnp.argmin(scores, axis=0).astype(jnp.int32)


def _sc_gather(wflat_hbm, idx_hbm, codes_hbm, w_v, idx_v, codes_v):
    wid = lax.axis_index("s") * _NC + lax.axis_index("c")
    b0 = wid * _BPW
    pltpu.sync_copy(wflat_hbm, w_v)
    pltpu.sync_copy(idx_hbm.at[pl.ds(b0, _BPW)], idx_v)

    @plsc.parallel_loop(0, _T // _L, unroll=2)
    def _body(g):
        sl = pl.ds(g * _L, _L)
        for bb in range(_BPW):
            base = idx_v[bb, sl] * _C
            for c in range(_C):
                codes_v[bb, c, sl] = plsc.load_gather(w_v, [base + c])

    pltpu.sync_copy(codes_v, codes_hbm.at[pl.ds(b0, _BPW)])


def kernel(x, W):
    xr = x.reshape(_B, _C, _T)
    idx3 = pl.pallas_call(
        _vq_idx_body,
        grid=(_B // 2,),
        in_specs=[
            pl.BlockSpec((2, _C, _T), lambda b: (b, 0, 0)),
            pl.BlockSpec((_K, _C), lambda b: (0, 0)),
        ],
        out_specs=pl.BlockSpec((2, 1, _T), lambda b: (b, 0, 0)),
        out_shape=jax.ShapeDtypeStruct((_B, 1, _T), jnp.int32),
        scratch_shapes=[pltpu.VMEM((_K, 1), jnp.float32)],
    )(xr, W)
    idx2 = idx3.reshape(_B, _T)

    gather = functools.partial(
        pl.kernel,
        out_type=jax.ShapeDtypeStruct((_B, _C, _T), jnp.float32),
        mesh=plsc.VectorSubcoreMesh(core_axis_name="c", subcore_axis_name="s"),
        scratch_types=[
            pltpu.VMEM((_C * _K,), jnp.float32),
            pltpu.VMEM((_BPW, _T), jnp.int32),
            pltpu.VMEM((_BPW, _C, _T), jnp.float32),
        ],
        compiler_params=pltpu.CompilerParams(
            needs_layout_passes=False, use_tc_tiling_on_sc=False),
    )(_sc_gather)
    codes3 = gather(W.reshape(-1), idx2)

    codes = codes3.reshape(x.shape)
    indices = idx2.reshape(_B, 32, 32)
    return codes, indices


# speedup vs baseline: 1.1002x; 1.0210x over previous
import functools

import jax
import jax.numpy as jnp
from jax import lax
from jax.experimental import pallas as pl
from jax.experimental.pallas import tpu as pltpu
from jax.experimental.pallas import tpu_sc as plsc

_B, _C, _K, _T = 64, 32, 1024, 1024
_NC, _NS, _L = 2, 16, 16
_NW = _NC * _NS
_BPW = _B // _NW


def _vq_idx_body(x_ref, w_ref, idx_ref, wn_ref):
    @pl.when(pl.program_id(0) == 0)
    def _():
        w = w_ref[...]
        wn_ref[...] = 0.5 * jnp.sum(w * w, axis=1, keepdims=True)

    for i in range(4):
        xb = x_ref[i]
        prod = jnp.dot(w_ref[...], xb,
                       preferred_element_type=jnp.float32)
        scores = wn_ref[...] - prod
        idx_ref[i, 0] = jnp.argmin(scores, axis=0).astype(jnp.int32)


def _sc_gather(wflat_hbm, idx_hbm, codes_hbm, w_v, idx_v, codes_v):
    wid = lax.axis_index("s") * _NC + lax.axis_index("c")
    b0 = wid * _BPW
    pltpu.sync_copy(wflat_hbm, w_v)
    pltpu.sync_copy(idx_hbm.at[pl.ds(b0, _BPW)], idx_v)

    @plsc.parallel_loop(0, _T // _L, unroll=2)
    def _body(g):
        sl = pl.ds(g * _L, _L)
        for bb in range(_BPW):
            base = idx_v[bb, sl] * _C
            for c in range(_C):
                codes_v[bb, c, sl] = plsc.load_gather(w_v, [base + c])

    pltpu.sync_copy(codes_v, codes_hbm.at[pl.ds(b0, _BPW)])


def kernel(x, W):
    xr = x.reshape(_B, _C, _T)
    idx3 = pl.pallas_call(
        _vq_idx_body,
        grid=(_B // 4,),
        in_specs=[
            pl.BlockSpec((4, _C, _T), lambda b: (b, 0, 0)),
            pl.BlockSpec((_K, _C), lambda b: (0, 0)),
        ],
        out_specs=pl.BlockSpec((4, 1, _T), lambda b: (b, 0, 0)),
        out_shape=jax.ShapeDtypeStruct((_B, 1, _T), jnp.int32),
        scratch_shapes=[pltpu.VMEM((_K, 1), jnp.float32)],
    )(xr, W)
    idx2 = idx3.reshape(_B, _T)

    gather = functools.partial(
        pl.kernel,
        out_type=jax.ShapeDtypeStruct((_B, _C, _T), jnp.float32),
        mesh=plsc.VectorSubcoreMesh(core_axis_name="c", subcore_axis_name="s"),
        scratch_types=[
            pltpu.VMEM((_C * _K,), jnp.float32),
            pltpu.VMEM((_BPW, _T), jnp.int32),
            pltpu.VMEM((_BPW, _C, _T), jnp.float32),
        ],
        compiler_params=pltpu.CompilerParams(
            needs_layout_passes=False, use_tc_tiling_on_sc=False),
    )(_sc_gather)
    codes3 = gather(W.reshape(-1), idx2)

    codes = codes3.reshape(x.shape)
    indices = idx2.reshape(_B, 32, 32)
    return codes, indices


# speedup vs baseline: 1.1113x; 1.0101x over previous
import functools

import jax
import jax.numpy as jnp
from jax import lax
from jax.experimental import pallas as pl
from jax.experimental.pallas import tpu as pltpu
from jax.experimental.pallas import tpu_sc as plsc

_B, _C, _K, _T = 64, 32, 1024, 1024
_NC, _NS, _L = 2, 16, 16
_NW = _NC * _NS
_BPW = _B // _NW


def _vq_idx_body(x_ref, w_ref, idx_ref, wn_ref):
    @pl.when(pl.program_id(0) == 0)
    def _():
        w = w_ref[...]
        wn_ref[...] = 0.5 * jnp.sum(w * w, axis=1, keepdims=True)

    for i in range(8):
        xb = x_ref[i]
        prod = jnp.dot(w_ref[...], xb,
                       preferred_element_type=jnp.float32)
        scores = wn_ref[...] - prod
        idx_ref[i, 0] = jnp.argmin(scores, axis=0).astype(jnp.int32)


def _sc_gather(wflat_hbm, idx_hbm, codes_hbm, w_v, idx_v, codes_v):
    wid = lax.axis_index("s") * _NC + lax.axis_index("c")
    b0 = wid * _BPW
    pltpu.sync_copy(wflat_hbm, w_v)
    pltpu.sync_copy(idx_hbm.at[pl.ds(b0, _BPW)], idx_v)

    @plsc.parallel_loop(0, _T // _L, unroll=2)
    def _body(g):
        sl = pl.ds(g * _L, _L)
        for bb in range(_BPW):
            base = idx_v[bb, sl] * _C
            for c in range(_C):
                codes_v[bb, c, sl] = plsc.load_gather(w_v, [base + c])

    pltpu.sync_copy(codes_v, codes_hbm.at[pl.ds(b0, _BPW)])


def kernel(x, W):
    xr = x.reshape(_B, _C, _T)
    idx3 = pl.pallas_call(
        _vq_idx_body,
        grid=(_B // 8,),
        in_specs=[
            pl.BlockSpec((8, _C, _T), lambda b: (b, 0, 0)),
            pl.BlockSpec((_K, _C), lambda b: (0, 0)),
        ],
        out_specs=pl.BlockSpec((8, 1, _T), lambda b: (b, 0, 0)),
        out_shape=jax.ShapeDtypeStruct((_B, 1, _T), jnp.int32),
        scratch_shapes=[pltpu.VMEM((_K, 1), jnp.float32)],
    )(xr, W)
    idx2 = idx3.reshape(_B, _T)

    gather = functools.partial(
        pl.kernel,
        out_type=jax.ShapeDtypeStruct((_B, _C, _T), jnp.float32),
        mesh=plsc.VectorSubcoreMesh(core_axis_name="c", subcore_axis_name="s"),
        scratch_types=[
            pltpu.VMEM((_C * _K,), jnp.float32),
            pltpu.VMEM((_BPW, _T), jnp.int32),
            pltpu.VMEM((_BPW, _C, _T), jnp.float32),
        ],
        compiler_params=pltpu.CompilerParams(
            needs_layout_passes=False, use_tc_tiling_on_sc=False),
    )(_sc_gather)
    codes3 = gather(W.reshape(-1), idx2)

    codes = codes3.reshape(x.shape)
    indices = idx2.reshape(_B, 32, 32)
    return codes, indices


# speedup vs baseline: 1.1149x; 1.0032x over previous
import functools

import jax
import jax.numpy as jnp
from jax import lax
from jax.experimental import pallas as pl
from jax.experimental.pallas import tpu as pltpu
from jax.experimental.pallas import tpu_sc as plsc

_B, _C, _K, _T = 64, 32, 1024, 1024
_NC, _NS, _L = 2, 16, 16
_NW = _NC * _NS
_BPW = _B // _NW


def _vq_idx_body(x_ref, w_ref, idx_ref, wn_ref):
    @pl.when(pl.program_id(0) == 0)
    def _():
        w = w_ref[...]
        wn_ref[...] = 0.5 * jnp.sum(w * w, axis=1, keepdims=True)

    for i in range(16):
        xb = x_ref[i]
        prod = jnp.dot(w_ref[...], xb,
                       preferred_element_type=jnp.float32)
        scores = wn_ref[...] - prod
        idx_ref[i, 0] = jnp.argmin(scores, axis=0).astype(jnp.int32)


def _sc_gather(wflat_hbm, idx_hbm, codes_hbm, w_v, idx_v, codes_v):
    wid = lax.axis_index("s") * _NC + lax.axis_index("c")
    b0 = wid * _BPW
    pltpu.sync_copy(wflat_hbm, w_v)
    pltpu.sync_copy(idx_hbm.at[pl.ds(b0, _BPW)], idx_v)

    @plsc.parallel_loop(0, _T // _L, unroll=2)
    def _body(g):
        sl = pl.ds(g * _L, _L)
        for bb in range(_BPW):
            base = idx_v[bb, sl] * _C
            for c in range(_C):
                codes_v[bb, c, sl] = plsc.load_gather(w_v, [base + c])

    pltpu.sync_copy(codes_v, codes_hbm.at[pl.ds(b0, _BPW)])


def kernel(x, W):
    xr = x.reshape(_B, _C, _T)
    idx3 = pl.pallas_call(
        _vq_idx_body,
        grid=(_B // 16,),
        in_specs=[
            pl.BlockSpec((16, _C, _T), lambda b: (b, 0, 0)),
            pl.BlockSpec((_K, _C), lambda b: (0, 0)),
        ],
        out_specs=pl.BlockSpec((16, 1, _T), lambda b: (b, 0, 0)),
        out_shape=jax.ShapeDtypeStruct((_B, 1, _T), jnp.int32),
        scratch_shapes=[pltpu.VMEM((_K, 1), jnp.float32)],
    )(xr, W)
    idx2 = idx3.reshape(_B, _T)

    gather = functools.partial(
        pl.kernel,
        out_type=jax.ShapeDtypeStruct((_B, _C, _T), jnp.float32),
        mesh=plsc.VectorSubcoreMesh(core_axis_name="c", subcore_axis_name="s"),
        scratch_types=[
            pltpu.VMEM((_C * _K,), jnp.float32),
            pltpu.VMEM((_BPW, _T), jnp.int32),
            pltpu.VMEM((_BPW, _C, _T), jnp.float32),
        ],
        compiler_params=pltpu.CompilerParams(
            needs_layout_passes=False, use_tc_tiling_on_sc=False),
    )(_sc_gather)
    codes3 = gather(W.reshape(-1), idx2)

    codes = codes3.reshape(x.shape)
    indices = idx2.reshape(_B, 32, 32)
    return codes, indices
